# baseline (speedup 1.0000x reference)
import functools

import jax
import jax.numpy as jnp
from jax import lax
from jax.experimental import pallas as pl
from jax.experimental.pallas import tpu as pltpu
from jax.experimental.pallas import tpu_sc as plsc

_V = 1_000_000
_D = 64
_DP = 128
_VL = _V // 2
_B = 4096 * 200
_BATCH = 4096
_SEQ = 200
_NC, _NS, _L = 2, 16, 16
_NW = _NC * _NS
_B_PER_W = _B // _NW
_CHUNK = 128
_N_CHUNKS = _B_PER_W // _CHUNK
_NBUF = 4
_N_ROUNDS = _N_CHUNKS // _NBUF
_SCALE = 8.0

_mesh = plsc.VectorSubcoreMesh(core_axis_name="c", subcore_axis_name="s")


@functools.partial(
    pl.kernel,
    out_type=jax.ShapeDtypeStruct((_NW, _N_CHUNKS, _CHUNK, _DP), jnp.float32),
    mesh=_mesh,
    scratch_types=[
        pltpu.VMEM((_N_CHUNKS, _CHUNK), jnp.int32),
        [pltpu.VMEM((_CHUNK, _DP), jnp.float32) for _ in range(_NBUF)],
        [pltpu.SemaphoreType.DMA for _ in range(_NBUF)],
        [pltpu.SemaphoreType.DMA for _ in range(_NBUF)],
    ],
    compiler_params=pltpu.CompilerParams(use_tc_tiling_on_sc=True),
)
def _gather_sc(idx_hbm, table_hbm, out_hbm, idx_v, rows, gsems, osems):
    wid = lax.axis_index("s") * _NC + lax.axis_index("c")
    pltpu.sync_copy(idx_hbm.at[wid], idx_v)

    def start_gather(j, b):
        pltpu.async_copy(table_hbm.at[idx_v.at[j]], rows[b], gsems[b])

    def wait_gather(j, b):
        pltpu.make_async_copy(table_hbm.at[idx_v.at[j]], rows[b], gsems[b]).wait()

    def start_out(j, b):
        pltpu.async_copy(rows[b], out_hbm.at[wid, j], osems[b])

    def wait_out(j, b):
        pltpu.make_async_copy(rows[b], out_hbm.at[wid, j], osems[b]).wait()

    for b in range(_NBUF):
        start_gather(b, b)

    def round_body(g, _):
        j0 = g * _NBUF
        for b in range(_NBUF):
            wait_gather(j0 + b, b)
            start_out(j0 + b, b)
        for b in range(_NBUF):
            wait_out(j0 + b, b)
            start_gather(j0 + _NBUF + b, b)
        return 0

    lax.fori_loop(0, _N_ROUNDS - 1, round_body, 0)

    j0 = (_N_ROUNDS - 1) * _NBUF
    for b in range(_NBUF):
        wait_gather(j0 + b, b)
        start_out(j0 + b, b)
    for b in range(_NBUF):
        wait_out(j0 + b, b)


_SBS = 4


def _finish_body(a_ref, p_ref, o_ref):
    for k in range(_SBS):
        a = a_ref[k * _BATCH:(k + 1) * _BATCH, :]
        p = p_ref[0, k][:, None].astype(jnp.float32)
        sel = a[:, :_D] * (1.0 - p) + a[:, _D:] * p
        o_ref[k] = jnp.transpose(sel, (1, 0)) * _SCALE


_finish_tc = pl.pallas_call(
    _finish_body,
    grid=(_SEQ // _SBS,),
    in_specs=[
        pl.BlockSpec((_SBS * _BATCH, _DP), lambda i: (i, 0)),
        pl.BlockSpec((1, _SBS, _BATCH), lambda i: (i, 0, 0)),
    ],
    out_specs=pl.BlockSpec((_SBS, _D, _BATCH), lambda i: (i, 0, 0)),
    out_shape=jax.ShapeDtypeStruct((_SEQ, _D, _BATCH), jnp.float32),
)


def kernel(x, embed):
    table = embed.reshape(_VL, _DP)
    xt = x.T.astype(jnp.int32)
    idx = (xt >> 1).reshape(_NW, _N_CHUNKS, _CHUNK)
    par = (xt & 1).reshape(_SEQ // _SBS, _SBS, _BATCH)
    rows = _gather_sc(idx, table)
    phys = _finish_tc(rows.reshape(_B, _DP), par)
    return jnp.transpose(phys, (2, 0, 1))


# speedup vs baseline: 1.1886x; 1.1886x over previous
import functools

import jax
import jax.numpy as jnp
from jax import lax
from jax.experimental import pallas as pl
from jax.experimental.pallas import tpu as pltpu
from jax.experimental.pallas import tpu_sc as plsc

_V = 1_000_000
_D = 64
_DP = 128
_B = 4096 * 200
_BATCH = 4096
_SEQ = 200
_NC, _NS, _L = 2, 16, 16
_NW = _NC * _NS
_B_PER_W = _B // _NW
_CHUNK = 128
_N_CHUNKS = _B_PER_W // _CHUNK
_NBUF = 4
_N_ROUNDS = _N_CHUNKS // _NBUF
_SCALE = 8.0

_mesh = plsc.VectorSubcoreMesh(core_axis_name="c", subcore_axis_name="s")


@functools.partial(
    pl.kernel,
    out_type=jax.ShapeDtypeStruct((_NW, _N_CHUNKS, _CHUNK, _DP), jnp.float32),
    mesh=_mesh,
    scratch_types=[
        pltpu.VMEM((_N_CHUNKS, _CHUNK), jnp.int32),
        [pltpu.VMEM((_CHUNK, _DP), jnp.float32) for _ in range(_NBUF)],
        [pltpu.SemaphoreType.DMA for _ in range(_NBUF)],
        [pltpu.SemaphoreType.DMA for _ in range(_NBUF)],
    ],
    compiler_params=pltpu.CompilerParams(use_tc_tiling_on_sc=False),
)
def _gather_sc(idx_hbm, table_hbm, out_hbm, idx_v, rows, gsems, osems):
    wid = lax.axis_index("s") * _NC + lax.axis_index("c")
    pltpu.sync_copy(idx_hbm.at[wid], idx_v)

    def start_gather(j, b):
        pltpu.async_copy(table_hbm.at[idx_v.at[j]], rows[b], gsems[b])

    def wait_gather(j, b):
        pltpu.make_async_copy(table_hbm.at[idx_v.at[j]], rows[b], gsems[b]).wait()

    def start_out(j, b):
        pltpu.async_copy(rows[b], out_hbm.at[wid, j], osems[b])

    def wait_out(j, b):
        pltpu.make_async_copy(rows[b], out_hbm.at[wid, j], osems[b]).wait()

    for b in range(_NBUF):
        start_gather(b, b)

    def round_body(g, _):
        j0 = g * _NBUF
        for b in range(_NBUF):
            wait_gather(j0 + b, b)
            start_out(j0 + b, b)
        for b in range(_NBUF):
            wait_out(j0 + b, b)
            start_gather(j0 + _NBUF + b, b)
        return 0

    lax.fori_loop(0, _N_ROUNDS - 1, round_body, 0)

    j0 = (_N_ROUNDS - 1) * _NBUF
    for b in range(_NBUF):
        wait_gather(j0 + b, b)
        start_out(j0 + b, b)
    for b in range(_NBUF):
        wait_out(j0 + b, b)


_SBS = 4


def _finish_body(a_ref, o_ref):
    for k in range(_SBS):
        sub = a_ref[k * _BATCH:(k + 1) * _BATCH, :_D]
        o_ref[k] = jnp.transpose(sub, (1, 0)) * _SCALE


_finish_tc = pl.pallas_call(
    _finish_body,
    grid=(_SEQ // _SBS,),
    in_specs=[pl.BlockSpec((_SBS * _BATCH, _DP), lambda i: (i, 0))],
    out_specs=pl.BlockSpec((_SBS, _D, _BATCH), lambda i: (i, 0, 0)),
    out_shape=jax.ShapeDtypeStruct((_SEQ, _D, _BATCH), jnp.float32),
)


def kernel(x, embed):
    table = jnp.pad(embed, ((0, 0), (0, _DP - _D)))
    idx = x.T.reshape(_NW, _N_CHUNKS, _CHUNK).astype(jnp.int32)
    rows = _gather_sc(idx, table)
    phys = _finish_tc(rows.reshape(_B, _DP))
    return jnp.transpose(phys, (2, 0, 1))


# speedup vs baseline: 1.1904x; 1.0015x over previous
import functools

import jax
import jax.numpy as jnp
from jax import lax
from jax.experimental import pallas as pl
from jax.experimental.pallas import tpu as pltpu
from jax.experimental.pallas import tpu_sc as plsc

_V = 1_000_000
_D = 64
_DP = 128
_B = 4096 * 200
_BATCH = 4096
_SEQ = 200
_NC, _NS, _L = 2, 16, 16
_NW = _NC * _NS
_B_PER_W = _B // _NW
_CHUNK = 128
_N_CHUNKS = _B_PER_W // _CHUNK
_NBUF = 4
_N_ROUNDS = _N_CHUNKS // _NBUF
_SCALE = 8.0

_mesh = plsc.VectorSubcoreMesh(
    core_axis_name="c", subcore_axis_name="s",
    num_cores=_NC, num_subcores=_NS,
)


@functools.partial(
    pl.kernel,
    out_type=jax.ShapeDtypeStruct((_NW, _N_CHUNKS, _CHUNK, _DP), jnp.float32),
    mesh=_mesh,
    scratch_types=[
        pltpu.VMEM((_N_CHUNKS, _CHUNK), jnp.int32),
        [pltpu.VMEM((_CHUNK, _DP), jnp.float32) for _ in range(_NBUF)],
        [pltpu.SemaphoreType.DMA for _ in range(_NBUF)],
        [pltpu.SemaphoreType.DMA for _ in range(_NBUF)],
    ],
    compiler_params=pltpu.CompilerParams(use_tc_tiling_on_sc=False),
)
def _gather_sc(idx_hbm, table_hbm, out_hbm, idx_v, rows, gsems, osems):
    wid = lax.axis_index("s") * _NC + lax.axis_index("c")
    pltpu.sync_copy(idx_hbm.at[wid], idx_v)

    def start_gather(j, b):
        pltpu.async_copy(table_hbm.at[idx_v.at[j]], rows[b], gsems[b])

    def wait_gather(j, b):
        pltpu.make_async_copy(table_hbm.at[idx_v.at[j]], rows[b], gsems[b]).wait()

    def start_out(j, b):
        pltpu.async_copy(rows[b], out_hbm.at[wid, j], osems[b])

    def wait_out(j, b):
        pltpu.make_async_copy(rows[b], out_hbm.at[wid, j], osems[b]).wait()

    for b in range(_NBUF):
        start_gather(b, b)

    def round_body(g, _):
        j0 = g * _NBUF
        for b in range(_NBUF):
            wait_gather(j0 + b, b)
            start_out(j0 + b, b)
        for b in range(_NBUF):
            wait_out(j0 + b, b)
            start_gather(j0 + _NBUF + b, b)
        return 0

    lax.fori_loop(0, _N_ROUNDS - 1, round_body, 0)

    j0 = (_N_ROUNDS - 1) * _NBUF
    for b in range(_NBUF):
        wait_gather(j0 + b, b)
        start_out(j0 + b, b)
    for b in range(_NBUF):
        wait_out(j0 + b, b)


_SBS = 4


def _finish_body(a_ref, o_ref):
    for k in range(_SBS):
        sub = a_ref[k * _BATCH:(k + 1) * _BATCH, :_D]
        o_ref[k] = jnp.transpose(sub, (1, 0)) * _SCALE


_finish_tc = pl.pallas_call(
    _finish_body,
    grid=(_SEQ // _SBS,),
    in_specs=[pl.BlockSpec((_SBS * _BATCH, _DP), lambda i: (i, 0))],
    out_specs=pl.BlockSpec((_SBS, _D, _BATCH), lambda i: (i, 0, 0)),
    out_shape=jax.ShapeDtypeStruct((_SEQ, _D, _BATCH), jnp.float32),
)


def kernel(x, embed):
    table = jnp.pad(embed, ((0, 0), (0, _DP - _D)))
    idx = x.T.reshape(_NW, _N_CHUNKS, _CHUNK).astype(jnp.int32)
    rows = _gather_sc(idx, table)
    phys = _finish_tc(rows.reshape(_B, _DP))
    return jnp.transpose(phys, (2, 0, 1))
